# Initial kernel scaffold; baseline (speedup 1.0000x reference)
import jax
import jax.numpy as jnp
from jax.experimental import pallas as pl


def kernel(x, edge_index, W1, as1, ad1, b1, W2, as2, ad2, b2, W3, as3, ad3, b3):
    raise NotImplementedError("write your pallas kernel here")


# speedup vs baseline: 38.0852x; 38.0852x over previous
import functools

import jax
import jax.numpy as jnp
from jax import lax
from jax.experimental import pallas as pl
from jax.experimental.pallas import tpu as pltpu
from jax.experimental.pallas import tpu_sc as plsc

N = 10000
E = 320000
ETOT = E + N
NPAD = 10240
JUNK = 10100
D = 128
D3 = 48
NCLS = 40
NH = 8

NC = 2
NS = 16
NWORK = NC * NS
CE = 128
KCH = (ETOT + NWORK * CE - 1) // (NWORK * CE)
EP = NWORK * CE * KCH
RPT = NPAD // NS

BLK = 512
GRID = NPAD // BLK

_f32 = jnp.float32


def _dense1_body(x_ref, w_ref, am_ref, h_ref, aa_ref):
    h = jnp.dot(x_ref[...], w_ref[...], preferred_element_type=_f32)
    h_ref[...] = h
    aa_ref[...] = jnp.dot(h, am_ref[...], preferred_element_type=_f32)


def _dense_mid_body(p0_ref, p1_ref, b_ref, w_ref, am_ref, h_ref, aa_ref):
    hin = jnp.maximum(p0_ref[...] + p1_ref[...] + b_ref[...], 0.0)
    h = jnp.dot(hin, w_ref[...], preferred_element_type=_f32)
    h_ref[...] = h
    aa_ref[...] = jnp.dot(h, am_ref[...], preferred_element_type=_f32)


def _tc_dense1(xp, W, am):
    return pl.pallas_call(
        _dense1_body,
        grid=(GRID,),
        in_specs=[
            pl.BlockSpec((BLK, D), lambda i: (i, 0)),
            pl.BlockSpec((D, D), lambda i: (0, 0)),
            pl.BlockSpec((D, 16), lambda i: (0, 0)),
        ],
        out_shape=[
            jax.ShapeDtypeStruct((NPAD, D), _f32),
            jax.ShapeDtypeStruct((NPAD, 16), _f32),
        ],
        out_specs=[
            pl.BlockSpec((BLK, D), lambda i: (i, 0)),
            pl.BlockSpec((BLK, 16), lambda i: (i, 0)),
        ],
    )(xp, W, am)


def _tc_dense_mid(p0, p1, b, W, am, dout):
    return pl.pallas_call(
        _dense_mid_body,
        grid=(GRID,),
        in_specs=[
            pl.BlockSpec((BLK, D), lambda i: (i, 0)),
            pl.BlockSpec((BLK, D), lambda i: (i, 0)),
            pl.BlockSpec((1, D), lambda i: (0, 0)),
            pl.BlockSpec((D, dout), lambda i: (0, 0)),
            pl.BlockSpec((dout, 16), lambda i: (0, 0)),
        ],
        out_shape=[
            jax.ShapeDtypeStruct((NPAD, dout), _f32),
            jax.ShapeDtypeStruct((NPAD, 16), _f32),
        ],
        out_specs=[
            pl.BlockSpec((BLK, dout), lambda i: (i, 0)),
            pl.BlockSpec((BLK, 16), lambda i: (i, 0)),
        ],
    )(p0, p1, b, W, am)


def _rden_body(d0_ref, d1_ref, o_ref):
    o_ref[...] = 1.0 / (d0_ref[...] + d1_ref[...] + 1e-16)


def _tc_rden(d0, d1):
    return pl.pallas_call(
        _rden_body,
        grid=(GRID,),
        in_specs=[
            pl.BlockSpec((BLK, 16), lambda i: (i, 0)),
            pl.BlockSpec((BLK, 16), lambda i: (i, 0)),
        ],
        out_shape=jax.ShapeDtypeStruct((NPAD, 16), _f32),
        out_specs=pl.BlockSpec((BLK, 16), lambda i: (i, 0)),
    )(d0, d1)


def _final_body(p0_ref, p1_ref, b_ref, o_ref):
    o_ref[...] = p0_ref[...] + p1_ref[...] + b_ref[...]


def _tc_final(p0, p1, b):
    return pl.pallas_call(
        _final_body,
        grid=(GRID,),
        in_specs=[
            pl.BlockSpec((BLK, D3), lambda i: (i, 0)),
            pl.BlockSpec((BLK, D3), lambda i: (i, 0)),
            pl.BlockSpec((1, D3), lambda i: (0, 0)),
        ],
        out_shape=jax.ShapeDtypeStruct((NPAD, D3), _f32),
        out_specs=pl.BlockSpec((BLK, D3), lambda i: (i, 0)),
    )(p0, p1, b)


def _dyn_gather(v, idx):
    return lax.gather(
        v, idx[:, None],
        lax.GatherDimensionNumbers(
            offset_dims=(), collapsed_slice_dims=(0,), start_index_map=(0,)),
        (1,), mode=lax.GatherScatterMode.PROMISE_IN_BOUNDS)


@functools.cache
def _mesh():
    return plsc.VectorSubcoreMesh(core_axis_name="c", subcore_axis_name="s")


def _edge_w(sa_ref, da_ref, i, qshift):
    s_v = sa_ref[i, :]
    d_v = _dyn_gather(da_ref[i, :], qshift)
    e = s_v + d_v
    e = jnp.where(e >= 0.0, e, 0.2 * e)
    return jnp.exp(e)


def _sc_pass_a(aa, src1, dst1, z16):
    @functools.partial(
        pl.kernel, mesh=_mesh(),
        compiler_params=pltpu.CompilerParams(use_tc_tiling_on_sc=False),
        out_type=jax.ShapeDtypeStruct((NC, NPAD, 16), _f32),
        scratch_types=[
            pltpu.VMEM((CE,), jnp.int32),
            pltpu.VMEM((CE,), jnp.int32),
            pltpu.VMEM((CE, 16), _f32),
            pltpu.VMEM((CE, 16), _f32),
            pltpu.VMEM((CE, 16), _f32),
            pltpu.VMEM_SHARED((NPAD, 16), _f32),
            pltpu.SemaphoreType.DMA,
            pltpu.SemaphoreType.DMA,
        ],
    )
    def k(aa_hbm, src_hbm, dst_hbm, z_hbm, dp_hbm,
          srcb, dstb, sa, da, wb, acc, sem1, sem2):
        cid = lax.axis_index("c")
        sid = lax.axis_index("s")
        wid = sid * NC + cid
        pltpu.sync_copy(z_hbm.at[pl.ds(0, CE)], wb)

        def zinit(j, _):
            pltpu.sync_copy(wb, acc.at[pl.ds(sid * RPT + j * CE, CE)])
            return 0

        lax.fori_loop(0, RPT // CE, zinit, 0)
        plsc.subcore_barrier()

        lane = lax.iota(jnp.int32, 16)
        qshift = (lane + 8) % 16

        def chunk(kk, _):
            base = (wid * KCH + kk) * CE
            pltpu.sync_copy(src_hbm.at[pl.ds(base, CE)], srcb)
            pltpu.sync_copy(dst_hbm.at[pl.ds(base, CE)], dstb)
            cp1 = pltpu.async_copy(aa_hbm.at[srcb], sa, sem1)
            cp2 = pltpu.async_copy(aa_hbm.at[dstb], da, sem2)
            cp1.wait()
            cp2.wait()

            def inner(i, _2):
                wb[i, :] = _edge_w(sa, da, i, qshift)
                return 0

            lax.fori_loop(0, CE, inner, 0)
            pltpu.sync_copy(wb, acc.at[dstb], add=True)
            return 0

        lax.fori_loop(0, KCH, chunk, 0)
        plsc.subcore_barrier()

        def wback(j, _):
            r = pl.ds(sid * RPT + j * CE, CE)
            pltpu.sync_copy(acc.at[r], wb)
            pltpu.sync_copy(wb, dp_hbm.at[cid].at[r])
            return 0

        lax.fori_loop(0, RPT // CE, wback, 0)

    return k(aa, src1, dst1, z16)


def _make_pass_b(dd, nv, head_per_vec):
    @functools.partial(
        pl.kernel, mesh=_mesh(),
        compiler_params=pltpu.CompilerParams(use_tc_tiling_on_sc=False),
        out_type=jax.ShapeDtypeStruct((NC, NPAD, dd), _f32),
        scratch_types=[
            pltpu.VMEM((CE,), jnp.int32),
            pltpu.VMEM((CE,), jnp.int32),
            pltpu.VMEM((CE, dd), _f32),
            pltpu.VMEM((CE, 16), _f32),
            pltpu.VMEM((CE, 16), _f32),
            pltpu.VMEM((CE, 16), _f32),
            pltpu.VMEM_SHARED((NPAD, dd), _f32),
            pltpu.SemaphoreType.DMA,
            pltpu.SemaphoreType.DMA,
            pltpu.SemaphoreType.DMA,
            pltpu.SemaphoreType.DMA,
        ],
    )
    def k(h_hbm, aa_hbm, rd_hbm, src_hbm, dst_hbm, z_hbm, out_hbm,
          srcb, dstb, rowsb, sa, da, rdb, acc, sem1, sem2, sem3, sem4):
        cid = lax.axis_index("c")
        sid = lax.axis_index("s")
        wid = sid * NC + cid
        pltpu.sync_copy(z_hbm.at[pl.ds(0, CE)], rowsb)

        def zinit(j, _):
            pltpu.sync_copy(rowsb, acc.at[pl.ds(sid * RPT + j * CE, CE)])
            return 0

        lax.fori_loop(0, RPT // CE, zinit, 0)
        plsc.subcore_barrier()

        lane = lax.iota(jnp.int32, 16)
        qshift = (lane + 8) % 16

        def chunk(kk, _):
            base = (wid * KCH + kk) * CE
            pltpu.sync_copy(src_hbm.at[pl.ds(base, CE)], srcb)
            pltpu.sync_copy(dst_hbm.at[pl.ds(base, CE)], dstb)
            cp1 = pltpu.async_copy(h_hbm.at[srcb], rowsb, sem1)
            cp2 = pltpu.async_copy(aa_hbm.at[srcb], sa, sem2)
            cp3 = pltpu.async_copy(aa_hbm.at[dstb], da, sem3)
            cp4 = pltpu.async_copy(rd_hbm.at[dstb], rdb, sem4)
            cp1.wait()
            cp2.wait()
            cp3.wait()
            cp4.wait()

            def inner(i, _2):
                av = _edge_w(sa, da, i, qshift) * rdb[i, :]
                for v in range(nv):
                    s = _dyn_gather(av, jnp.full((16,), head_per_vec[v],
                                                 jnp.int32))
                    rowsb[i, pl.ds(16 * v, 16)] = rowsb[i, pl.ds(16 * v, 16)] * s
                return 0

            lax.fori_loop(0, CE, inner, 0)
            pltpu.sync_copy(rowsb, acc.at[dstb], add=True)
            return 0

        lax.fori_loop(0, KCH, chunk, 0)
        plsc.subcore_barrier()

        def wback(j, _):
            r = pl.ds(sid * RPT + j * CE, CE)
            pltpu.sync_copy(acc.at[r], rowsb)
            pltpu.sync_copy(rowsb, out_hbm.at[cid].at[r])
            return 0

        lax.fori_loop(0, RPT // CE, wback, 0)

    return k


def _sc_pass_b(h, aa, rden, src1, dst1, zdd, dd, nhead):
    nv = dd // 16
    head_per_vec = tuple(v if nhead > 1 else 0 for v in range(nv))
    return _make_pass_b(dd, nv, head_per_vec)(h, aa, rden, src1, dst1, zdd)


def _attn_mat(a_s, a_d, din):
    h_, fo = a_s.shape
    r = jnp.arange(h_ * fo)
    hd = (r // fo).astype(jnp.int32)
    m = jnp.zeros((din, 16), _f32)
    m = m.at[r, hd].set(a_s.reshape(-1))
    m = m.at[r, hd + 8].set(a_d.reshape(-1))
    return m


def _gat_layer_edges(h, aa, src1, dst1, z16, zdd, dd, nhead):
    dp = _sc_pass_a(aa, src1, dst1, z16)
    rden = _tc_rden(dp[0], dp[1])
    outp = _sc_pass_b(h, aa, rden, src1, dst1, zdd, dd, nhead)
    return outp[0], outp[1]


def kernel(x, edge_index, W1, as1, ad1, b1, W2, as2, ad2, b2,
           W3, as3, ad3, b3):
    xp = jnp.zeros((NPAD, D), _f32).at[:N].set(x)
    loop = jnp.arange(N, dtype=jnp.int32)
    npad_e = EP - ETOT
    src = jnp.concatenate(
        [edge_index[0], loop, jnp.zeros((npad_e,), jnp.int32)])
    dst = jnp.concatenate(
        [edge_index[1], loop, jnp.full((npad_e,), JUNK, jnp.int32)])

    am1 = _attn_mat(as1, ad1, D)
    am2 = _attn_mat(as2, ad2, D)
    W3p = jnp.zeros((D, D3), _f32).at[:, :NCLS].set(W3)
    am3 = _attn_mat(as3, ad3, D3)[: D3]
    b3p = jnp.zeros((1, D3), _f32).at[0, :NCLS].set(b3)

    z16 = jnp.zeros((NPAD, 16), _f32)
    z128 = jnp.zeros((NPAD, D), _f32)
    z48 = jnp.zeros((NPAD, D3), _f32)

    h1, aa1 = _tc_dense1(xp, W1, am1)
    p1a, p1b = _gat_layer_edges(h1, aa1, src, dst, z16, z128, D, NH)
    h2, aa2 = _tc_dense_mid(p1a, p1b, b1.reshape(1, D), W2, am2, D)
    p2a, p2b = _gat_layer_edges(h2, aa2, src, dst, z16, z128, D, NH)
    h3, aa3 = _tc_dense_mid(p2a, p2b, b2.reshape(1, D), W3p, am3, D3)
    p3a, p3b = _gat_layer_edges(h3, aa3, src, dst, z16, z48, D3, 1)
    out = _tc_final(p3a, p3b, b3p)
    return out[:N, :NCLS]


# speedup vs baseline: 48.9612x; 1.2856x over previous
import functools

import jax
import jax.numpy as jnp
from jax import lax
from jax.experimental import pallas as pl
from jax.experimental.pallas import tpu as pltpu
from jax.experimental.pallas import tpu_sc as plsc

N = 10000
E = 320000
ETOT = E + N
NPAD = 10240
JUNK = 10100
D = 128
D3 = 48
NCLS = 40
NH = 8

NC = 2
NS = 16
NWORK = NC * NS
CE = 128
KCH = (ETOT + NWORK * CE - 1) // (NWORK * CE)
EP = NWORK * CE * KCH
RPT = NPAD // NS

BLK = 512
GRID = NPAD // BLK

_f32 = jnp.float32


def _dense1_body(x_ref, w_ref, am_ref, h_ref, aa_ref):
    h = jnp.dot(x_ref[...], w_ref[...], preferred_element_type=_f32)
    h_ref[...] = h
    aa_ref[...] = jnp.dot(h, am_ref[...], preferred_element_type=_f32)


def _dense_mid_body(p0_ref, p1_ref, rx_ref, b_ref, w_ref, am_ref,
                    h_ref, aa_ref):
    hin = jnp.maximum(
        (p0_ref[...] + p1_ref[...]) * rx_ref[...] + b_ref[...], 0.0)
    h = jnp.dot(hin, w_ref[...], preferred_element_type=_f32)
    h_ref[...] = h
    aa_ref[...] = jnp.dot(h, am_ref[...], preferred_element_type=_f32)


def _tc_dense1(xp, W, am):
    return pl.pallas_call(
        _dense1_body,
        grid=(GRID,),
        in_specs=[
            pl.BlockSpec((BLK, D), lambda i: (i, 0)),
            pl.BlockSpec((D, D), lambda i: (0, 0)),
            pl.BlockSpec((D, 16), lambda i: (0, 0)),
        ],
        out_shape=[
            jax.ShapeDtypeStruct((NPAD, D), _f32),
            jax.ShapeDtypeStruct((NPAD, 16), _f32),
        ],
        out_specs=[
            pl.BlockSpec((BLK, D), lambda i: (i, 0)),
            pl.BlockSpec((BLK, 16), lambda i: (i, 0)),
        ],
    )(xp, W, am)


def _tc_dense_mid(p0, p1, rx, b, W, am, dout):
    return pl.pallas_call(
        _dense_mid_body,
        grid=(GRID,),
        in_specs=[
            pl.BlockSpec((BLK, D), lambda i: (i, 0)),
            pl.BlockSpec((BLK, D), lambda i: (i, 0)),
            pl.BlockSpec((BLK, D), lambda i: (i, 0)),
            pl.BlockSpec((1, D), lambda i: (0, 0)),
            pl.BlockSpec((D, dout), lambda i: (0, 0)),
            pl.BlockSpec((dout, 16), lambda i: (0, 0)),
        ],
        out_shape=[
            jax.ShapeDtypeStruct((NPAD, dout), _f32),
            jax.ShapeDtypeStruct((NPAD, 16), _f32),
        ],
        out_specs=[
            pl.BlockSpec((BLK, dout), lambda i: (i, 0)),
            pl.BlockSpec((BLK, 16), lambda i: (i, 0)),
        ],
    )(p0, p1, rx, b, W, am)


def _rdenx_body(d0_ref, d1_ref, e8_ref, o_ref):
    dx = jnp.dot(d0_ref[...] + d1_ref[...], e8_ref[...],
                 preferred_element_type=_f32)
    o_ref[...] = 1.0 / (dx + 1e-16)


def _tc_rdenx(d0, d1, e8, dout):
    return pl.pallas_call(
        _rdenx_body,
        grid=(GRID,),
        in_specs=[
            pl.BlockSpec((BLK, 16), lambda i: (i, 0)),
            pl.BlockSpec((BLK, 16), lambda i: (i, 0)),
            pl.BlockSpec((16, dout), lambda i: (0, 0)),
        ],
        out_shape=jax.ShapeDtypeStruct((NPAD, dout), _f32),
        out_specs=pl.BlockSpec((BLK, dout), lambda i: (i, 0)),
    )(d0, d1, e8)


def _final_body(p0_ref, p1_ref, rx_ref, b_ref, o_ref):
    o_ref[...] = (p0_ref[...] + p1_ref[...]) * rx_ref[...] + b_ref[...]


def _tc_final(p0, p1, rx, b):
    return pl.pallas_call(
        _final_body,
        grid=(GRID,),
        in_specs=[
            pl.BlockSpec((BLK, D3), lambda i: (i, 0)),
            pl.BlockSpec((BLK, D3), lambda i: (i, 0)),
            pl.BlockSpec((BLK, D3), lambda i: (i, 0)),
            pl.BlockSpec((1, D3), lambda i: (0, 0)),
        ],
        out_shape=jax.ShapeDtypeStruct((NPAD, D3), _f32),
        out_specs=pl.BlockSpec((BLK, D3), lambda i: (i, 0)),
    )(p0, p1, rx, b)


def _dyn_gather(v, idx):
    return lax.gather(
        v, idx[:, None],
        lax.GatherDimensionNumbers(
            offset_dims=(), collapsed_slice_dims=(0,), start_index_map=(0,)),
        (1,), mode=lax.GatherScatterMode.PROMISE_IN_BOUNDS)


@functools.cache
def _mesh():
    return plsc.VectorSubcoreMesh(core_axis_name="c", subcore_axis_name="s")


def _edge_w(sa_ref, da_ref, i, qshift):
    s_v = sa_ref[i, :]
    d_v = _dyn_gather(da_ref[i, :], qshift)
    e = s_v + d_v
    e = jnp.where(e >= 0.0, e, 0.2 * e)
    return jnp.exp(e)


def _sc_pass_a(aa, src1, dst1, z16):
    @functools.partial(
        pl.kernel, mesh=_mesh(),
        compiler_params=pltpu.CompilerParams(use_tc_tiling_on_sc=False),
        out_type=[
            jax.ShapeDtypeStruct((NC, NPAD, 16), _f32),
            jax.ShapeDtypeStruct((EP, 16), _f32),
        ],
        scratch_types=[
            pltpu.VMEM((CE,), jnp.int32),
            pltpu.VMEM((CE,), jnp.int32),
            pltpu.VMEM((CE, 16), _f32),
            pltpu.VMEM((CE, 16), _f32),
            pltpu.VMEM((CE, 16), _f32),
            pltpu.VMEM_SHARED((NPAD, 16), _f32),
            pltpu.SemaphoreType.DMA,
            pltpu.SemaphoreType.DMA,
        ],
    )
    def k(aa_hbm, src_hbm, dst_hbm, z_hbm, dp_hbm, w_hbm,
          srcb, dstb, sa, da, wb, acc, sem1, sem2):
        cid = lax.axis_index("c")
        sid = lax.axis_index("s")
        wid = sid * NC + cid
        pltpu.sync_copy(z_hbm.at[pl.ds(0, CE)], wb)

        def zinit(j, _):
            pltpu.sync_copy(wb, acc.at[pl.ds(sid * RPT + j * CE, CE)])
            return 0

        lax.fori_loop(0, RPT // CE, zinit, 0)
        plsc.subcore_barrier()

        lane = lax.iota(jnp.int32, 16)
        qshift = (lane + 8) % 16

        def chunk(kk, _):
            base = (wid * KCH + kk) * CE
            pltpu.sync_copy(src_hbm.at[pl.ds(base, CE)], srcb)
            pltpu.sync_copy(dst_hbm.at[pl.ds(base, CE)], dstb)
            cp1 = pltpu.async_copy(aa_hbm.at[srcb], sa, sem1)
            cp2 = pltpu.async_copy(aa_hbm.at[dstb], da, sem2)
            cp1.wait()
            cp2.wait()

            def inner(i, _2):
                wb[i, :] = _edge_w(sa, da, i, qshift)
                return 0

            lax.fori_loop(0, CE, inner, 0)
            pltpu.sync_copy(wb, w_hbm.at[pl.ds(base, CE)])
            pltpu.sync_copy(wb, acc.at[dstb], add=True)
            return 0

        lax.fori_loop(0, KCH, chunk, 0)
        plsc.subcore_barrier()

        def wback(j, _):
            r = pl.ds(sid * RPT + j * CE, CE)
            pltpu.sync_copy(acc.at[r], wb)
            pltpu.sync_copy(wb, dp_hbm.at[cid].at[r])
            return 0

        lax.fori_loop(0, RPT // CE, wback, 0)

    return k(aa, src1, dst1, z16)


def _make_pass_b(dd, nv, head_per_vec):
    @functools.partial(
        pl.kernel, mesh=_mesh(),
        compiler_params=pltpu.CompilerParams(use_tc_tiling_on_sc=False),
        out_type=jax.ShapeDtypeStruct((NC, NPAD, dd), _f32),
        scratch_types=[
            pltpu.VMEM((CE,), jnp.int32),
            pltpu.VMEM((CE,), jnp.int32),
            pltpu.VMEM((CE, dd), _f32),
            pltpu.VMEM((CE, 16), _f32),
            pltpu.VMEM_SHARED((NPAD, dd), _f32),
            pltpu.SemaphoreType.DMA,
            pltpu.SemaphoreType.DMA,
        ],
    )
    def k(h_hbm, w_hbm, src_hbm, dst_hbm, z_hbm, out_hbm,
          srcb, dstb, rowsb, wb, acc, sem1, sem2):
        cid = lax.axis_index("c")
        sid = lax.axis_index("s")
        wid = sid * NC + cid
        pltpu.sync_copy(z_hbm.at[pl.ds(0, CE)], rowsb)

        def zinit(j, _):
            pltpu.sync_copy(rowsb, acc.at[pl.ds(sid * RPT + j * CE, CE)])
            return 0

        lax.fori_loop(0, RPT // CE, zinit, 0)
        plsc.subcore_barrier()

        def chunk(kk, _):
            base = (wid * KCH + kk) * CE
            pltpu.sync_copy(src_hbm.at[pl.ds(base, CE)], srcb)
            pltpu.sync_copy(dst_hbm.at[pl.ds(base, CE)], dstb)
            cp1 = pltpu.async_copy(h_hbm.at[srcb], rowsb, sem1)
            cp2 = pltpu.async_copy(w_hbm.at[pl.ds(base, CE)], wb, sem2)
            cp1.wait()
            cp2.wait()

            def inner(i, _2):
                av = wb[i, :]
                for v in range(nv):
                    s = _dyn_gather(av, jnp.full((16,), head_per_vec[v],
                                                 jnp.int32))
                    rowsb[i, pl.ds(16 * v, 16)] = rowsb[i, pl.ds(16 * v, 16)] * s
                return 0

            lax.fori_loop(0, CE, inner, 0)
            pltpu.sync_copy(rowsb, acc.at[dstb], add=True)
            return 0

        lax.fori_loop(0, KCH, chunk, 0)
        plsc.subcore_barrier()

        def wback(j, _):
            r = pl.ds(sid * RPT + j * CE, CE)
            pltpu.sync_copy(acc.at[r], rowsb)
            pltpu.sync_copy(rowsb, out_hbm.at[cid].at[r])
            return 0

        lax.fori_loop(0, RPT // CE, wback, 0)

    return k


def _sc_pass_b(h, w, src1, dst1, zdd, dd, nhead):
    nv = dd // 16
    head_per_vec = tuple(v if nhead > 1 else 0 for v in range(nv))
    return _make_pass_b(dd, nv, head_per_vec)(h, w, src1, dst1, zdd)


def _attn_mat(a_s, a_d, din):
    h_, fo = a_s.shape
    r = jnp.arange(h_ * fo)
    hd = (r // fo).astype(jnp.int32)
    m = jnp.zeros((din, 16), _f32)
    m = m.at[r, hd].set(a_s.reshape(-1))
    m = m.at[r, hd + 8].set(a_d.reshape(-1))
    return m


def _gat_layer_edges(h, aa, src1, dst1, z16, zdd, e8, dd, nhead):
    dp, w = _sc_pass_a(aa, src1, dst1, z16)
    rx = _tc_rdenx(dp[0], dp[1], e8, dd)
    outp = _sc_pass_b(h, w, src1, dst1, zdd, dd, nhead)
    return outp[0], outp[1], rx


def kernel(x, edge_index, W1, as1, ad1, b1, W2, as2, ad2, b2,
           W3, as3, ad3, b3):
    xp = jnp.zeros((NPAD, D), _f32).at[:N].set(x)
    loop = jnp.arange(N, dtype=jnp.int32)
    npad_e = EP - ETOT
    src = jnp.concatenate(
        [edge_index[0], loop, jnp.zeros((npad_e,), jnp.int32)])
    dst = jnp.concatenate(
        [edge_index[1], loop, jnp.full((npad_e,), JUNK, jnp.int32)])

    am1 = _attn_mat(as1, ad1, D)
    am2 = _attn_mat(as2, ad2, D)
    W3p = jnp.zeros((D, D3), _f32).at[:, :NCLS].set(W3)
    am3 = _attn_mat(as3, ad3, D3)[: D3]
    b3p = jnp.zeros((1, D3), _f32).at[0, :NCLS].set(b3)

    z16 = jnp.zeros((NPAD, 16), _f32)
    z128 = jnp.zeros((NPAD, D), _f32)
    z48 = jnp.zeros((NPAD, D3), _f32)
    cols = jnp.arange(D)
    e8 = jnp.zeros((16, D), _f32).at[cols // 16, cols].set(1.0)
    e83 = jnp.zeros((16, D3), _f32).at[0, :].set(1.0)

    h1, aa1 = _tc_dense1(xp, W1, am1)
    p1a, p1b, rx1 = _gat_layer_edges(h1, aa1, src, dst, z16, z128, e8, D, NH)
    h2, aa2 = _tc_dense_mid(p1a, p1b, rx1, b1.reshape(1, D), W2, am2, D)
    p2a, p2b, rx2 = _gat_layer_edges(h2, aa2, src, dst, z16, z128, e8, D, NH)
    h3, aa3 = _tc_dense_mid(p2a, p2b, rx2, b2.reshape(1, D), W3p, am3, D3)
    p3a, p3b, rx3 = _gat_layer_edges(h3, aa3, src, dst, z16, z48, e83, D3, 1)
    out = _tc_final(p3a, p3b, rx3, b3p)
    return out[:N, :NCLS]


# speedup vs baseline: 58.4851x; 1.1945x over previous
import functools

import jax
import jax.numpy as jnp
from jax import lax
from jax.experimental import pallas as pl
from jax.experimental.pallas import tpu as pltpu
from jax.experimental.pallas import tpu_sc as plsc

N = 10000
E = 320000
ETOT = E + N
NPAD = 10240
JUNK = 10100
D = 128
DW = D + 16
D3 = 48
NCLS = 40
NV = D // 16

NC = 2
NS = 16
NWORK = NC * NS
CE = 56
KCH = 188
EP = NWORK * CE * KCH
RPT = NPAD // NS
RQ, RR = RPT // CE, RPT % CE

BLK = 512
GRID = NPAD // BLK

_f32 = jnp.float32


def _dense1_body(x_ref, w_ref, am_ref, h_ref, aa_ref):
    h = jnp.dot(x_ref[...], w_ref[...], preferred_element_type=_f32)
    aa = jnp.dot(h, am_ref[...], preferred_element_type=_f32)
    h_ref[:, :D] = h
    h_ref[:, D:] = aa
    aa_ref[...] = aa


def _make_dense_mid(dout):
    def body(p0_ref, p1_ref, e8_ref, b_ref, w_ref, am_ref, h_ref, aa_ref):
        dx = jnp.dot(p0_ref[:, D:] + p1_ref[:, D:], e8_ref[...],
                     preferred_element_type=_f32)
        hin = jnp.maximum(
            (p0_ref[:, :D] + p1_ref[:, :D]) / (dx + 1e-16) + b_ref[...],
            0.0)
        h = jnp.dot(hin, w_ref[...], preferred_element_type=_f32)
        aa = jnp.dot(h, am_ref[...], preferred_element_type=_f32)
        h_ref[:, :dout] = h
        if dout < D:
            h_ref[:, dout:D] = jnp.zeros((BLK, D - dout), _f32)
        h_ref[:, D:] = aa
        aa_ref[...] = aa
    return body


def _tc_dense1(xp, W, am):
    return pl.pallas_call(
        _dense1_body,
        grid=(GRID,),
        in_specs=[
            pl.BlockSpec((BLK, D), lambda i: (i, 0)),
            pl.BlockSpec((D, D), lambda i: (0, 0)),
            pl.BlockSpec((D, 16), lambda i: (0, 0)),
        ],
        out_shape=[
            jax.ShapeDtypeStruct((NPAD, DW), _f32),
            jax.ShapeDtypeStruct((NPAD, 16), _f32),
        ],
        out_specs=[
            pl.BlockSpec((BLK, DW), lambda i: (i, 0)),
            pl.BlockSpec((BLK, 16), lambda i: (i, 0)),
        ],
    )(xp, W, am)


def _tc_dense_mid(p0, p1, e8, b, W, am, dout):
    return pl.pallas_call(
        _make_dense_mid(dout),
        grid=(GRID,),
        in_specs=[
            pl.BlockSpec((BLK, DW), lambda i: (i, 0)),
            pl.BlockSpec((BLK, DW), lambda i: (i, 0)),
            pl.BlockSpec((16, D), lambda i: (0, 0)),
            pl.BlockSpec((1, D), lambda i: (0, 0)),
            pl.BlockSpec((D, dout), lambda i: (0, 0)),
            pl.BlockSpec((dout, 16), lambda i: (0, 0)),
        ],
        out_shape=[
            jax.ShapeDtypeStruct((NPAD, DW), _f32),
            jax.ShapeDtypeStruct((NPAD, 16), _f32),
        ],
        out_specs=[
            pl.BlockSpec((BLK, DW), lambda i: (i, 0)),
            pl.BlockSpec((BLK, 16), lambda i: (i, 0)),
        ],
    )(p0, p1, e8, b, W, am)


def _final_body(p0_ref, p1_ref, e8_ref, b_ref, o_ref):
    dx = jnp.dot(p0_ref[:, D:] + p1_ref[:, D:], e8_ref[...],
                 preferred_element_type=_f32)
    o_ref[...] = (p0_ref[:, :D3] + p1_ref[:, :D3]) / (dx + 1e-16) + b_ref[...]


def _tc_final(p0, p1, e83, b):
    return pl.pallas_call(
        _final_body,
        grid=(GRID,),
        in_specs=[
            pl.BlockSpec((BLK, DW), lambda i: (i, 0)),
            pl.BlockSpec((BLK, DW), lambda i: (i, 0)),
            pl.BlockSpec((16, D3), lambda i: (0, 0)),
            pl.BlockSpec((1, D3), lambda i: (0, 0)),
        ],
        out_shape=jax.ShapeDtypeStruct((NPAD, D3), _f32),
        out_specs=pl.BlockSpec((BLK, D3), lambda i: (i, 0)),
    )(p0, p1, e83, b)


def _dyn_gather(v, idx):
    return lax.gather(
        v, idx[:, None],
        lax.GatherDimensionNumbers(
            offset_dims=(), collapsed_slice_dims=(0,), start_index_map=(0,)),
        (1,), mode=lax.GatherScatterMode.PROMISE_IN_BOUNDS)


@functools.cache
def _mesh():
    return plsc.VectorSubcoreMesh(core_axis_name="c", subcore_axis_name="s")


@functools.cache
def _fused_kernel():
    @functools.partial(
        pl.kernel, mesh=_mesh(),
        compiler_params=pltpu.CompilerParams(use_tc_tiling_on_sc=False),
        out_type=jax.ShapeDtypeStruct((NC, NPAD, DW), _f32),
        scratch_types=[
            pltpu.VMEM((NV, 16), jnp.int32),
            pltpu.VMEM((CE,), jnp.int32), pltpu.VMEM((CE,), jnp.int32),
            pltpu.VMEM((CE,), jnp.int32), pltpu.VMEM((CE,), jnp.int32),
            pltpu.VMEM((CE,), jnp.int32), pltpu.VMEM((CE,), jnp.int32),
            pltpu.VMEM((CE,), jnp.int32), pltpu.VMEM((CE,), jnp.int32),
            pltpu.VMEM((CE, 16), _f32), pltpu.VMEM((CE, 16), _f32),
            pltpu.VMEM((CE, 16), _f32), pltpu.VMEM((CE, 16), _f32),
            pltpu.VMEM((CE, DW), _f32), pltpu.VMEM((CE, DW), _f32),
            pltpu.VMEM((CE, DW), _f32), pltpu.VMEM((CE, DW), _f32),
            pltpu.VMEM_SHARED((NPAD, DW), _f32),
            pltpu.SemaphoreType.DMA, pltpu.SemaphoreType.DMA,
            pltpu.SemaphoreType.DMA, pltpu.SemaphoreType.DMA,
            pltpu.SemaphoreType.DMA, pltpu.SemaphoreType.DMA,
            pltpu.SemaphoreType.DMA, pltpu.SemaphoreType.DMA,
            pltpu.SemaphoreType.DMA, pltpu.SemaphoreType.DMA,
            pltpu.SemaphoreType.DMA, pltpu.SemaphoreType.DMA,
        ],
    )
    def k(aa_hbm, h_hbm, src_hbm, dst_hbm, hv_hbm, out_hbm,
          hvb, sb0, sb1, sb2, sb3, db0, db1, db2, db3,
          da0, da1, da2, da3, rb0, rb1, rb2, rb3, acc,
          si0, si1, si2, si3, sg0, sg1, sg2, sg3, so0, so1, so2, so3):
        cid = lax.axis_index("c")
        sid = lax.axis_index("s")
        wid = sid * NC + cid
        wbase = wid * KCH
        sbs = (sb0, sb1, sb2, sb3)
        dbs = (db0, db1, db2, db3)
        sis = (si0, si1, si2, si3)
        das = (da0, da1, da2, da3)
        rbs = (rb0, rb1, rb2, rb3)
        sgs = (sg0, sg1, sg2, sg3)
        sos = (so0, so1, so2, so3)

        pltpu.sync_copy(hv_hbm, hvb)

        zv = jnp.zeros((16,), _f32)

        def zrow(i, _):
            for v in range(NV + 1):
                rb0[i, pl.ds(16 * v, 16)] = zv
            return 0

        lax.fori_loop(0, CE, zrow, 0)

        def zinit(j, _):
            pltpu.sync_copy(rb0, acc.at[pl.ds(sid * RPT + j * CE, CE)])
            return 0

        lax.fori_loop(0, RQ, zinit, 0)
        pltpu.sync_copy(rb0.at[pl.ds(0, RR)],
                        acc.at[pl.ds(sid * RPT + RQ * CE, RR)])
        plsc.subcore_barrier()

        lane = lax.iota(jnp.int32, 16)
        qshift = (lane + 8) % 16
        hvs = [hvb[v, :] for v in range(NV)]

        def issue_idx(kk, q):
            base = (wbase + kk) * CE
            pltpu.async_copy(src_hbm.at[pl.ds(base, CE)], sbs[q], sis[q])
            pltpu.async_copy(dst_hbm.at[pl.ds(base, CE)], dbs[q], sis[q])

        def wait_idx(kk, q):
            base = (wbase + kk) * CE
            pltpu.make_async_copy(
                src_hbm.at[pl.ds(base, CE)], sbs[q], sis[q]).wait()
            pltpu.make_async_copy(
                dst_hbm.at[pl.ds(base, CE)], dbs[q], sis[q]).wait()

        def issue_data(q):
            pltpu.async_copy(aa_hbm.at[dbs[q]], das[q], sgs[q])
            pltpu.async_copy(h_hbm.at[sbs[q]], rbs[q], sgs[q])

        def wait_data(q):
            pltpu.make_async_copy(aa_hbm.at[dbs[q]], das[q], sgs[q]).wait()
            pltpu.make_async_copy(h_hbm.at[sbs[q]], rbs[q], sgs[q]).wait()

        def compute(q):
            da, rb = das[q], rbs[q]

            def inner(i, _2):
                s_v = rb[i, pl.ds(D, 16)]
                d_v = _dyn_gather(da[i, :], qshift)
                e = s_v + d_v
                e = jnp.where(e >= 0.0, e, 0.2 * e)
                w = jnp.exp(e)
                for v in range(NV):
                    s = _dyn_gather(w, hvs[v])
                    rb[i, pl.ds(16 * v, 16)] = rb[i, pl.ds(16 * v, 16)] * s
                rb[i, pl.ds(D, 16)] = w
                return 0

            lax.fori_loop(0, CE, inner, 0)

        def outs(q):
            pltpu.async_copy(rbs[q], acc.at[dbs[q]], sos[q], add=True)

        def wait_outs(q):
            pltpu.make_async_copy(rbs[q], acc.at[dbs[q]], sos[q]).wait()

        def half(kk, q, first=False, tail=False, last=False):
            if not first:
                wait_outs((q + 2) % 4)
            if not (tail or last):
                issue_idx(kk + 2, (q + 2) % 4)
            if not last:
                wait_idx(kk + 1, (q + 1) % 4)
                issue_data((q + 1) % 4)
            wait_data(q)
            compute(q)
            outs(q)

        issue_idx(0, 0)
        issue_idx(1, 1)
        wait_idx(0, 0)
        issue_data(0)
        half(0, 0, first=True)
        half(1, 1, first=True)

        def quad(p, _):
            kk = 2 + 4 * p
            half(kk + 0, 2)
            half(kk + 1, 3)
            half(kk + 2, 0)
            half(kk + 3, 1)
            return 0

        lax.fori_loop(0, (KCH - 4) // 4, quad, 0)
        half(KCH - 2, 2, tail=True)
        half(KCH - 1, 3, last=True)
        wait_outs(2)
        wait_outs(3)
        plsc.subcore_barrier()

        def wback(j, _):
            r = pl.ds(sid * RPT + j * CE, CE)
            pltpu.sync_copy(acc.at[r], rb0)
            pltpu.sync_copy(rb0, out_hbm.at[cid].at[r])
            return 0

        lax.fori_loop(0, RQ, wback, 0)
        rr = pl.ds(sid * RPT + RQ * CE, RR)
        pltpu.sync_copy(acc.at[rr], rb0.at[pl.ds(0, RR)])
        pltpu.sync_copy(rb0.at[pl.ds(0, RR)], out_hbm.at[cid].at[rr])

    return k


def _sc_fused(aa, h, src1, dst1, hv):
    return _fused_kernel()(aa, h, src1, dst1, hv)


def _attn_mat(a_s, a_d, din):
    h_, fo = a_s.shape
    r = jnp.arange(h_ * fo)
    hd = (r // fo).astype(jnp.int32)
    m = jnp.zeros((din, 16), _f32)
    m = m.at[r, hd].set(a_s.reshape(-1))
    m = m.at[r, hd + 8].set(a_d.reshape(-1))
    return m


def kernel(x, edge_index, W1, as1, ad1, b1, W2, as2, ad2, b2,
           W3, as3, ad3, b3):
    xp = jnp.zeros((NPAD, D), _f32).at[:N].set(x)
    loop = jnp.arange(N, dtype=jnp.int32)
    npad_e = EP - ETOT
    src = jnp.concatenate(
        [edge_index[0], loop, jnp.zeros((npad_e,), jnp.int32)])
    dst = jnp.concatenate(
        [edge_index[1], loop, jnp.full((npad_e,), JUNK, jnp.int32)])

    am1 = _attn_mat(as1, ad1, D)
    am2 = _attn_mat(as2, ad2, D)
    W3p = jnp.zeros((D, D3), _f32).at[:, :NCLS].set(W3)
    am3 = _attn_mat(as3, ad3, D3)[: D3]
    b3p = jnp.zeros((1, D3), _f32).at[0, :NCLS].set(b3)

    cols = jnp.arange(D)
    e8 = jnp.zeros((16, D), _f32).at[cols // 16, cols].set(1.0)
    e83 = jnp.zeros((16, D3), _f32).at[0, :].set(1.0)
    hv12 = jnp.broadcast_to(
        jnp.arange(NV, dtype=jnp.int32)[:, None], (NV, 16))
    hv3 = jnp.zeros((NV, 16), jnp.int32)

    h1, aa1 = _tc_dense1(xp, W1, am1)
    op1 = _sc_fused(aa1, h1, src, dst, hv12)
    h2, aa2 = _tc_dense_mid(op1[0], op1[1], e8, b1.reshape(1, D),
                            W2, am2, D)
    op2 = _sc_fused(aa2, h2, src, dst, hv12)
    h3, aa3 = _tc_dense_mid(op2[0], op2[1], e8, b2.reshape(1, D),
                            W3p, am3, D3)
    op3 = _sc_fused(aa3, h3, src, dst, hv3)
    out = _tc_final(op3[0], op3[1], e83, b3p)
    return out[:N, :NCLS]
